# baseline (speedup 1.0000x reference)
import jax
import jax.numpy as jnp
from jax import lax
from jax.experimental import pallas as pl
from jax.experimental.pallas import tpu as pltpu
from jax.experimental.pallas import tpu_sc as plsc

TOTAL = 819200
B = 4096
V = 1000000
D = 64

NC = 2
NS = 16
NW = NC * NS
BAGS_PER_W = B // NW
CHUNK = 512
ROWS_PER_CHUNK = CHUNK // 128


def _sc_body(ids1d, offs, weight, out, idx_v, rows_v, acc_v, off_v, sem0,
             sem1):
    c = lax.axis_index("c")
    s = lax.axis_index("s")
    w = s * NC + c
    nbag = BAGS_PER_W

    pltpu.sync_copy(offs, off_v.at[pl.ds(0, B)])
    off_v[pl.ds(B, 16)] = jnp.full((16,), TOTAL, jnp.int32)
    wbase = w * nbag

    def _off(i):
        return off_v[pl.ds(wbase + i, 16)][0]

    zeros16 = jnp.zeros((16,), jnp.float32)
    def _zero(i, _):
        for j in range(4):
            acc_v[i, pl.ds(j * 16, 16)] = zeros16
        return 0
    lax.fori_loop(0, nbag, _zero, 0)

    tok_start = _off(0)
    tok_end = _off(nbag)
    row0 = tok_start // 128
    cbase = row0 * 128
    nch = lax.max(0, (tok_end - cbase + CHUNK - 1) // CHUNK)

    sems = (sem0, sem1)

    def load_ids(kc, p):
        lobuf = lax.min(cbase + kc * CHUNK, TOTAL - CHUNK)
        for j in range(ROWS_PER_CHUNK):
            pltpu.sync_copy(ids1d.at[pl.ds(lobuf + j * 128, 128)],
                            idx_v.at[p, j])

    def chunk_cps(p):
        return [pltpu.make_async_copy(weight.at[idx_v.at[p, j]],
                                      rows_v.at[p, pl.ds(j * 128, 128)],
                                      sems[p])
                for j in range(ROWS_PER_CHUNK)]

    def fire(p):
        for cp in chunk_cps(p):
            cp.start()

    def drain(p):
        for cp in chunk_cps(p):
            cp.wait()

    load_ids(jnp.int32(0), 0)
    fire(0)
    load_ids(jnp.int32(1), 1)
    fire(1)
    drain(0)

    def step(_, carry):
        k, lb = carry
        p = k & 1
        lo = cbase + k * CHUNK
        lobuf = lax.min(lo, TOTAL - CHUNK)
        chi = lax.min(lo + CHUNK, tok_end)
        sb = _off(lb)
        eb = _off(lb + 1)
        a = lax.max(sb, lo) - lobuf
        e = lax.min(eb, chi) - lobuf
        a0 = acc_v[lb, pl.ds(0, 16)]
        a1 = acc_v[lb, pl.ds(16, 16)]
        a2 = acc_v[lb, pl.ds(32, 16)]
        a3 = acc_v[lb, pl.ds(48, 16)]

        @plsc.parallel_loop(a, e, unroll=4, carry=(a0, a1, a2, a3))
        def tok(t, accs):
            b0, b1, b2, b3 = accs
            b0 = b0 + rows_v[p, t, pl.ds(0, 16)]
            b1 = b1 + rows_v[p, t, pl.ds(16, 16)]
            b2 = b2 + rows_v[p, t, pl.ds(32, 16)]
            b3 = b3 + rows_v[p, t, pl.ds(48, 16)]
            return b0, b1, b2, b3

        a0, a1, a2, a3 = tok
        acc_v[lb, pl.ds(0, 16)] = a0
        acc_v[lb, pl.ds(16, 16)] = a1
        acc_v[lb, pl.ds(32, 16)] = a2
        acc_v[lb, pl.ds(48, 16)] = a3
        finished = eb <= chi
        adv = jnp.logical_and(jnp.logical_not(finished), k + 1 < nch)

        @pl.when(jnp.logical_and(adv, p == 0))
        def _advance_even():
            load_ids(k + 2, 0)
            fire(0)
            drain(1)

        @pl.when(jnp.logical_and(adv, p == 1))
        def _advance_odd():
            load_ids(k + 2, 1)
            fire(1)
            drain(0)

        k_next = jnp.where(finished, k, k + 1)
        lb_next = jnp.where(finished, lb + 1, lb)
        return k_next, lb_next

    n_steps = nbag + lax.max(nch - 1, 0)
    lax.fori_loop(0, n_steps, step, (jnp.int32(0), jnp.int32(0)))

    pmax = lax.max(nch, 1) & 1

    @pl.when(pmax == 0)
    def _drain_even():
        drain(0)

    @pl.when(pmax == 1)
    def _drain_odd():
        drain(1)

    lane = lax.iota(jnp.int32, 16)
    perms = [lane ^ bit for bit in (1, 2, 4, 8)]

    def _rsqrt(v):
        y = lax.bitcast_convert_type(
            jnp.full((16,), 0x5F3759DF, jnp.int32)
            - (lax.bitcast_convert_type(v, jnp.int32) >> 1),
            jnp.float32)
        for _it in range(3):
            y = y * (1.5 - 0.5 * v * y * y)
        return y

    def norm_body(i, _):
        cnt = lax.max(_off(i + 1) - _off(i), 1)
        cntv = jnp.full((16,), 1.0, jnp.float32) * cnt.astype(jnp.float32)
        rc = _rsqrt(cntv)
        minv = rc * rc
        a0 = acc_v[i, pl.ds(0, 16)]
        a1 = acc_v[i, pl.ds(16, 16)]
        a2 = acc_v[i, pl.ds(32, 16)]
        a3 = acc_v[i, pl.ds(48, 16)]
        sq = a0 * a0 + a1 * a1 + a2 * a2 + a3 * a3
        for p in perms:
            _, shuf = plsc.sort_key_val(p, sq)
            sq = sq + shuf
        y = _rsqrt(sq)
        mnorm2 = sq * minv * minv
        factor = jnp.where(mnorm2 > 1e-24, y, minv * 1e12)
        acc_v[i, pl.ds(0, 16)] = a0 * factor
        acc_v[i, pl.ds(16, 16)] = a1 * factor
        acc_v[i, pl.ds(32, 16)] = a2 * factor
        acc_v[i, pl.ds(48, 16)] = a3 * factor
        return 0
    lax.fori_loop(0, nbag, norm_body, 0)

    pltpu.sync_copy(acc_v, out.at[pl.ds(w * nbag, nbag)])


@jax.jit
def _run(ids2d, offp, weight):
    mesh = plsc.VectorSubcoreMesh(core_axis_name="c", subcore_axis_name="s")
    kfn = pl.kernel(
        _sc_body,
        out_type=jax.ShapeDtypeStruct((B, D), jnp.float32),
        mesh=mesh,
        scratch_types=[
            pltpu.VMEM((2, ROWS_PER_CHUNK, 128), jnp.int32),
            pltpu.VMEM((2, CHUNK, D), jnp.float32),
            pltpu.VMEM((BAGS_PER_W, D), jnp.float32),
            pltpu.VMEM((B + 32,), jnp.int32),
            pltpu.SemaphoreType.DMA,
            pltpu.SemaphoreType.DMA,
        ],
        compiler_params=pltpu.CompilerParams(
            use_tc_tiling_on_sc=False, needs_layout_passes=False),
    )
    return kfn(ids2d, offp, weight)


def kernel(input_ids, offsets, weight):
    return _run(input_ids.astype(jnp.int32), offsets.astype(jnp.int32),
                weight)


# speedup vs baseline: 1.0986x; 1.0986x over previous
import jax
import jax.numpy as jnp
from jax import lax
from jax.experimental import pallas as pl
from jax.experimental.pallas import tpu as pltpu
from jax.experimental.pallas import tpu_sc as plsc

TOTAL = 819200
B = 4096
V = 1000000
D = 64

NC = 2
NS = 16
NW = NC * NS
BAGS_PER_W = B // NW
CHUNK = 512
ROWS_PER_CHUNK = CHUNK // 128


def _sc_body(ids2d, offp, weight, out, idx_v, rows_v, acc_v, off_v, sem0,
             sem1):
    c = lax.axis_index("c")
    s = lax.axis_index("s")
    w = s * NC + c
    nbag = BAGS_PER_W

    obase = pl.multiple_of(w * nbag, 8)
    pltpu.sync_copy(offp.at[pl.ds(obase, nbag + 8)],
                    off_v.at[pl.ds(0, nbag + 8)])

    def _off(i):
        return off_v[pl.ds(i, 16)][0]

    zeros16 = jnp.zeros((16,), jnp.float32)
    def _zero(i, _):
        for j in range(4):
            acc_v[i, pl.ds(j * 16, 16)] = zeros16
        return 0
    lax.fori_loop(0, nbag, _zero, 0)

    tok_start = _off(0)
    tok_end = _off(nbag)
    row0 = tok_start // 128
    cbase = row0 * 128
    nch = lax.max(0, (tok_end - cbase + CHUNK - 1) // CHUNK)

    sems = (sem0, sem1)

    def load_ids(kc, p):
        pltpu.sync_copy(ids2d.at[pl.ds(row0 + kc * ROWS_PER_CHUNK,
                                       ROWS_PER_CHUNK)], idx_v.at[p])

    def chunk_cps(p):
        return [pltpu.make_async_copy(weight.at[idx_v.at[p, j]],
                                      rows_v.at[p, pl.ds(j * 128, 128)],
                                      sems[p])
                for j in range(ROWS_PER_CHUNK)]

    def fire(p):
        for cp in chunk_cps(p):
            cp.start()

    def drain(p):
        for cp in chunk_cps(p):
            cp.wait()

    load_ids(jnp.int32(0), 0)
    fire(0)
    load_ids(jnp.int32(1), 1)
    fire(1)
    drain(0)

    def step(_, carry):
        k, lb = carry
        p = k & 1
        lo = cbase + k * CHUNK
        chi = lax.min(lo + CHUNK, tok_end)
        sb = _off(lb)
        eb = _off(lb + 1)
        a = lax.max(sb, lo) - lo
        e = lax.min(eb, chi) - lo
        a0 = acc_v[lb, pl.ds(0, 16)]
        a1 = acc_v[lb, pl.ds(16, 16)]
        a2 = acc_v[lb, pl.ds(32, 16)]
        a3 = acc_v[lb, pl.ds(48, 16)]

        @plsc.parallel_loop(a, e, unroll=4, carry=(a0, a1, a2, a3))
        def tok(t, accs):
            b0, b1, b2, b3 = accs
            b0 = b0 + rows_v[p, t, pl.ds(0, 16)]
            b1 = b1 + rows_v[p, t, pl.ds(16, 16)]
            b2 = b2 + rows_v[p, t, pl.ds(32, 16)]
            b3 = b3 + rows_v[p, t, pl.ds(48, 16)]
            return b0, b1, b2, b3

        a0, a1, a2, a3 = tok
        acc_v[lb, pl.ds(0, 16)] = a0
        acc_v[lb, pl.ds(16, 16)] = a1
        acc_v[lb, pl.ds(32, 16)] = a2
        acc_v[lb, pl.ds(48, 16)] = a3
        finished = eb <= chi
        adv = jnp.logical_and(jnp.logical_not(finished), k + 1 < nch)

        @pl.when(jnp.logical_and(adv, p == 0))
        def _advance_even():
            load_ids(k + 2, 0)
            fire(0)
            drain(1)

        @pl.when(jnp.logical_and(adv, p == 1))
        def _advance_odd():
            load_ids(k + 2, 1)
            fire(1)
            drain(0)

        k_next = jnp.where(finished, k, k + 1)
        lb_next = jnp.where(finished, lb + 1, lb)
        return k_next, lb_next

    n_steps = nbag + lax.max(nch - 1, 0)
    lax.fori_loop(0, n_steps, step, (jnp.int32(0), jnp.int32(0)))

    pmax = lax.max(nch, 1) & 1

    @pl.when(pmax == 0)
    def _drain_even():
        drain(0)

    @pl.when(pmax == 1)
    def _drain_odd():
        drain(1)

    lane = lax.iota(jnp.int32, 16)
    perms = [lane ^ bit for bit in (1, 2, 4, 8)]

    def _rsqrt(v):
        y = lax.bitcast_convert_type(
            jnp.full((16,), 0x5F3759DF, jnp.int32)
            - (lax.bitcast_convert_type(v, jnp.int32) >> 1),
            jnp.float32)
        for _it in range(3):
            y = y * (1.5 - 0.5 * v * y * y)
        return y

    def norm_body(i, _):
        cnt = lax.max(_off(i + 1) - _off(i), 1)
        cntv = jnp.full((16,), 1.0, jnp.float32) * cnt.astype(jnp.float32)
        rc = _rsqrt(cntv)
        minv = rc * rc
        a0 = acc_v[i, pl.ds(0, 16)]
        a1 = acc_v[i, pl.ds(16, 16)]
        a2 = acc_v[i, pl.ds(32, 16)]
        a3 = acc_v[i, pl.ds(48, 16)]
        sq = a0 * a0 + a1 * a1 + a2 * a2 + a3 * a3
        for p in perms:
            _, shuf = plsc.sort_key_val(p, sq)
            sq = sq + shuf
        y = _rsqrt(sq)
        mnorm2 = sq * minv * minv
        factor = jnp.where(mnorm2 > 1e-24, y, minv * 1e12)
        acc_v[i, pl.ds(0, 16)] = a0 * factor
        acc_v[i, pl.ds(16, 16)] = a1 * factor
        acc_v[i, pl.ds(32, 16)] = a2 * factor
        acc_v[i, pl.ds(48, 16)] = a3 * factor
        return 0
    lax.fori_loop(0, nbag, norm_body, 0)

    pltpu.sync_copy(acc_v, out.at[pl.ds(w * nbag, nbag)])


@jax.jit
def _run(ids2d, offp, weight):
    mesh = plsc.VectorSubcoreMesh(core_axis_name="c", subcore_axis_name="s")
    kfn = pl.kernel(
        _sc_body,
        out_type=jax.ShapeDtypeStruct((B, D), jnp.float32),
        mesh=mesh,
        scratch_types=[
            pltpu.VMEM((2, ROWS_PER_CHUNK, 128), jnp.int32),
            pltpu.VMEM((2, CHUNK, D), jnp.float32),
            pltpu.VMEM((BAGS_PER_W, D), jnp.float32),
            pltpu.VMEM((BAGS_PER_W + 24,), jnp.int32),
            pltpu.SemaphoreType.DMA,
            pltpu.SemaphoreType.DMA,
        ],
        compiler_params=pltpu.CompilerParams(
            use_tc_tiling_on_sc=False, needs_layout_passes=False),
    )
    return kfn(ids2d, offp, weight)


def kernel(input_ids, offsets, weight):
    input_ids = input_ids.astype(jnp.int32)
    offsets = offsets.astype(jnp.int32)
    ids_pad = jnp.concatenate(
        [input_ids, jnp.zeros((CHUNK,), jnp.int32)]).reshape(-1, 128)
    offp = jnp.concatenate(
        [offsets, jnp.full((8,), TOTAL, jnp.int32)])
    return _run(ids_pad, offp, weight)


# speedup vs baseline: 1.0998x; 1.0011x over previous
import jax
import jax.numpy as jnp
from jax import lax
from jax.experimental import pallas as pl
from jax.experimental.pallas import tpu as pltpu
from jax.experimental.pallas import tpu_sc as plsc

TOTAL = 819200
B = 4096
V = 1000000
D = 64

NC = 2
NS = 16
NW = NC * NS
BAGS_PER_W = B // NW
CHUNK = 512
ROWS_PER_CHUNK = CHUNK // 128


def _sc_body(ids2d, offp, weight, out, idx_v, rows_v, acc_v, off_v, sem0,
             sem1):
    c = lax.axis_index("c")
    s = lax.axis_index("s")
    w = s * NC + c
    nbag = BAGS_PER_W

    obase = pl.multiple_of(w * nbag, 8)
    pltpu.sync_copy(offp.at[pl.ds(obase, nbag + 8)],
                    off_v.at[pl.ds(0, nbag + 8)])

    def _off(i):
        return off_v[pl.ds(i, 16)][0]

    zeros16 = jnp.zeros((16,), jnp.float32)
    def _zero(i, _):
        for j in range(4):
            acc_v[i, pl.ds(j * 16, 16)] = zeros16
        return 0
    lax.fori_loop(0, nbag, _zero, 0)

    tok_start = _off(0)
    tok_end = _off(nbag)
    row0 = tok_start // 128
    cbase = row0 * 128
    nch = lax.max(0, (tok_end - cbase + CHUNK - 1) // CHUNK)

    sems = (sem0, sem1)

    def load_ids(kc, p):
        pltpu.sync_copy(ids2d.at[pl.ds(row0 + kc * ROWS_PER_CHUNK,
                                       ROWS_PER_CHUNK)], idx_v.at[p])

    def chunk_cps(p):
        return [pltpu.make_async_copy(weight.at[idx_v.at[p, j]],
                                      rows_v.at[p, pl.ds(j * 128, 128)],
                                      sems[p])
                for j in range(ROWS_PER_CHUNK)]

    def fire(p):
        for cp in chunk_cps(p):
            cp.start()

    def drain(p):
        for cp in chunk_cps(p):
            cp.wait()

    load_ids(jnp.int32(0), 0)
    fire(0)
    load_ids(jnp.int32(1), 1)
    fire(1)
    drain(0)

    def step(_, carry):
        k, lb = carry
        p = k & 1
        lo = cbase + k * CHUNK
        chi = lax.min(lo + CHUNK, tok_end)
        sb = _off(lb)
        eb = _off(lb + 1)
        a = lax.max(sb, lo) - lo
        e = lax.min(eb, chi) - lo
        a0 = acc_v[lb, pl.ds(0, 16)]
        a1 = acc_v[lb, pl.ds(16, 16)]
        a2 = acc_v[lb, pl.ds(32, 16)]
        a3 = acc_v[lb, pl.ds(48, 16)]

        @plsc.parallel_loop(a, e, unroll=4, carry=(a0, a1, a2, a3))
        def tok(t, accs):
            b0, b1, b2, b3 = accs
            b0 = b0 + rows_v[p, t, pl.ds(0, 16)]
            b1 = b1 + rows_v[p, t, pl.ds(16, 16)]
            b2 = b2 + rows_v[p, t, pl.ds(32, 16)]
            b3 = b3 + rows_v[p, t, pl.ds(48, 16)]
            return b0, b1, b2, b3

        a0, a1, a2, a3 = tok
        acc_v[lb, pl.ds(0, 16)] = a0
        acc_v[lb, pl.ds(16, 16)] = a1
        acc_v[lb, pl.ds(32, 16)] = a2
        acc_v[lb, pl.ds(48, 16)] = a3
        finished = eb <= chi
        adv = jnp.logical_and(jnp.logical_not(finished), k + 1 < nch)

        @pl.when(jnp.logical_and(adv, p == 0))
        def _advance_even():
            load_ids(k + 2, 0)
            fire(0)
            drain(1)

        @pl.when(jnp.logical_and(adv, p == 1))
        def _advance_odd():
            load_ids(k + 2, 1)
            fire(1)
            drain(0)

        k_next = jnp.where(finished, k, k + 1)
        lb_next = jnp.where(finished, lb + 1, lb)
        return k_next, lb_next

    n_steps = nbag + lax.max(nch - 1, 0)
    lax.fori_loop(0, n_steps, step, (jnp.int32(0), jnp.int32(0)))

    pmax = lax.max(nch, 1) & 1

    @pl.when(pmax == 0)
    def _drain_even():
        drain(0)

    @pl.when(pmax == 1)
    def _drain_odd():
        drain(1)

    lane = lax.iota(jnp.int32, 16)
    perms = [lane ^ bit for bit in (1, 2, 4, 8)]

    def _rsqrt(v):
        y = lax.bitcast_convert_type(
            jnp.full((16,), 0x5F3759DF, jnp.int32)
            - (lax.bitcast_convert_type(v, jnp.int32) >> 1),
            jnp.float32)
        for _it in range(3):
            y = y * (1.5 - 0.5 * v * y * y)
        return y

    def norm_body(i, _):
        cnt = lax.max(_off(i + 1) - _off(i), 1)
        cntv = jnp.full((16,), 1.0, jnp.float32) * cnt.astype(jnp.float32)
        rc = _rsqrt(cntv)
        minv = rc * rc
        a0 = acc_v[i, pl.ds(0, 16)]
        a1 = acc_v[i, pl.ds(16, 16)]
        a2 = acc_v[i, pl.ds(32, 16)]
        a3 = acc_v[i, pl.ds(48, 16)]
        sq = a0 * a0 + a1 * a1 + a2 * a2 + a3 * a3
        for p in perms:
            _, shuf = plsc.sort_key_val(p, sq)
            sq = sq + shuf
        y = _rsqrt(sq)
        mnorm2 = sq * minv * minv
        factor = jnp.where(mnorm2 > 1e-24, y, minv * 1e12)
        acc_v[i, pl.ds(0, 16)] = a0 * factor
        acc_v[i, pl.ds(16, 16)] = a1 * factor
        acc_v[i, pl.ds(32, 16)] = a2 * factor
        acc_v[i, pl.ds(48, 16)] = a3 * factor
        return 0
    lax.fori_loop(0, nbag, norm_body, 0)

    pltpu.sync_copy(acc_v, out.at[pl.ds(w * nbag, nbag)])


@jax.jit
def _run(ids2d, offp, weight):
    mesh = plsc.VectorSubcoreMesh(core_axis_name="c", subcore_axis_name="s")
    kfn = pl.kernel(
        _sc_body,
        out_type=jax.ShapeDtypeStruct((B, D), jnp.float32),
        mesh=mesh,
        scratch_types=[
            pltpu.VMEM((2, ROWS_PER_CHUNK, 128), jnp.int32),
            pltpu.VMEM((2, CHUNK, D), jnp.float32),
            pltpu.VMEM((BAGS_PER_W, D), jnp.float32),
            pltpu.VMEM((BAGS_PER_W + 24,), jnp.int32),
            pltpu.SemaphoreType.DMA,
            pltpu.SemaphoreType.DMA,
        ],
        compiler_params=pltpu.CompilerParams(
            use_tc_tiling_on_sc=False, needs_layout_passes=False),
    )
    return kfn(ids2d, offp, weight)


def kernel(input_ids, offsets, weight):
    input_ids = input_ids.astype(jnp.int32)
    offsets = offsets.astype(jnp.int32)
    ids_pad = jnp.concatenate(
        [input_ids, jnp.zeros((2 * CHUNK,), jnp.int32)]).reshape(-1, 128)
    offp = jnp.concatenate(
        [offsets, jnp.full((8,), TOTAL, jnp.int32)])
    return _run(ids_pad, offp, weight)
